# baseline (speedup 1.0000x reference)
import functools

import jax
import jax.numpy as jnp
from jax import lax
from jax.experimental import pallas as pl
from jax.experimental.pallas import tpu as pltpu
from jax.experimental.pallas import tpu_sc as plsc

_N = 10000
_E = 320000
_D = 128
_ED = 16
_L = 3
_G = 64

_NW = 32
_NP = 10240
_ROWS = 320
_CAP = 20608
_SCCH = 128
_DCH = 40000
_EPW = _E // _NW
_GCH = 80


def _mesh():
    return plsc.VectorSubcoreMesh(core_axis_name="c", subcore_axis_name="s")


def _sc_params():
    return pltpu.CompilerParams(needs_layout_passes=False)


def _wid():
    return lax.axis_index("s") * 2 + lax.axis_index("c")


@functools.lru_cache(maxsize=None)
def _build_lists():
    @functools.partial(
        pl.kernel,
        mesh=_mesh(),
        compiler_params=_sc_params(),
        out_type=[
            jax.ShapeDtypeStruct((_NW, _CAP), jnp.int32),
            jax.ShapeDtypeStruct((_NW, _CAP), jnp.int32),
            jax.ShapeDtypeStruct((_NW, 16), jnp.int32),
        ],
        scratch_types=[
            pltpu.VMEM((_DCH,), jnp.int32),
            pltpu.VMEM((_CAP,), jnp.int32),
            pltpu.VMEM((_CAP,), jnp.int32),
            pltpu.VMEM((16,), jnp.int32),
        ],
    )
    def k(dst_hbm, ids_hbm, ldst_hbm, cnt_hbm, dchunk, ids_v, ldst_v, cntb):
        wid = _wid()
        lo = wid * _ROWS
        z16 = jnp.zeros((16,), jnp.int32)

        def zb(i, carry):
            ids_v[pl.ds(i * 16, 16)] = z16
            return carry

        lax.fori_loop(0, _CAP // 16, zb, 0)

        def chunk_body(cidx, cnt):
            pltpu.sync_copy(dst_hbm.at[pl.ds(cidx * _DCH, _DCH)], dchunk)

            def ibody(i, cnt):
                v = dchunk[pl.ds(i * 16, 16)]
                eid = (cidx * _DCH + i * 16) + lax.iota(jnp.int32, 16)
                m = (v >= lo) & (v < lo + _ROWS)
                mi = m.astype(jnp.int32)
                pos = cnt + plsc.cumsum(mi) - 1
                plsc.store_scatter(ids_v, [pos], eid, mask=m)
                plsc.store_scatter(ldst_v, [pos], v - lo, mask=m)
                return cnt + jnp.sum(mi)

            return lax.fori_loop(0, _DCH // 16, ibody, cnt)

        cnt = lax.fori_loop(0, _E // _DCH, chunk_body, jnp.int32(0))
        iota16 = lax.iota(jnp.int32, 16)
        for kpad in range(8):
            pos = cnt + kpad * 16 + iota16
            plsc.store_scatter(ids_v, [pos], jnp.zeros((16,), jnp.int32))
            plsc.store_scatter(ldst_v, [pos],
                               jnp.full((16,), _ROWS, jnp.int32))
        cntp = ((cnt + _SCCH - 1) // _SCCH) * _SCCH
        pltpu.sync_copy(ids_v, ids_hbm.at[wid])
        pltpu.sync_copy(ldst_v, ldst_hbm.at[wid])
        cntb[...] = jnp.full((16,), cntp, jnp.int32)
        pltpu.sync_copy(cntb, cnt_hbm.at[wid])

    return k


@functools.lru_cache(maxsize=None)
def _make_agg(with_max):
    @functools.partial(
        pl.kernel,
        mesh=_mesh(),
        compiler_params=_sc_params(),
        out_type=jax.ShapeDtypeStruct((_NP, _D), jnp.float32),
        scratch_types=[
            pltpu.VMEM((_ROWS + 1, _D), jnp.float32),
            pltpu.VMEM((_ROWS + 1, _D), jnp.float32),
            pltpu.VMEM((2, _SCCH), jnp.int32),
            pltpu.VMEM((_SCCH,), jnp.int32),
            pltpu.VMEM((2, _SCCH, _D), jnp.float32),
            pltpu.VMEM((16,), jnp.int32),
            pltpu.SemaphoreType.DMA,
            pltpu.SemaphoreType.DMA,
        ],
    )
    def k(hedge_hbm, ids_hbm, ldst_hbm, cnt_hbm, out_hbm,
          sacc, macc, idsb, ldstb, rows, cntb, sem0, sem1):
        wid = _wid()
        zf = jnp.zeros((16,), jnp.float32)

        def zb(r, carry):
            for cc in range(8):
                sacc[r, pl.ds(cc * 16, 16)] = zf
                if with_max:
                    macc[r, pl.ds(cc * 16, 16)] = zf
            return carry

        lax.fori_loop(0, _ROWS, zb, 0)

        pltpu.sync_copy(cnt_hbm.at[wid], cntb)
        lane = lax.iota(jnp.int32, 16)
        nch = jnp.sum(jnp.where(lane == 0, cntb[...], 0)) // _SCCH
        sems = (sem0, sem1)

        def issue(c, slot):
            pltpu.sync_copy(ids_hbm.at[wid, pl.ds(c * _SCCH, _SCCH)],
                            idsb.at[slot])
            pltpu.async_copy(hedge_hbm.at[idsb.at[slot]], rows.at[slot],
                             sems[slot])

        def process(c, slot):
            base = c * _SCCH
            pltpu.make_async_copy(hedge_hbm.at[idsb.at[slot]],
                                  rows.at[slot], sems[slot]).wait()
            pltpu.sync_copy(ldst_hbm.at[wid, pl.ds(base, _SCCH)], ldstb)

            def grp(g, carry):
                ld16 = ldstb[pl.ds(g * 16, 16)]
                jb = g * 16

                for t in range(16):
                    ld = ld16[t]
                    srow = sacc.at[ld]
                    mrow = macc.at[ld]
                    erow = rows.at[slot, jb + t]
                    for r in range(8):
                        a = erow[pl.ds(r * 16, 16)]
                        plsc.addupdate(srow.at[pl.ds(r * 16, 16)], a)
                        if with_max:
                            mrow[pl.ds(r * 16, 16)] = jnp.maximum(
                                mrow[pl.ds(r * 16, 16)], a)

                return carry

            lax.fori_loop(0, _SCCH // 16, grp, 0)

        @pl.when(nch > 0)
        def _():
            issue(0, 0)

        def pair(kk, carry):
            c0 = kk * 2

            @pl.when(c0 + 1 < nch)
            def _():
                issue(c0 + 1, 1)
            process(c0, 0)

            @pl.when(c0 + 1 < nch)
            def _():
                @pl.when(c0 + 2 < nch)
                def _():
                    issue(c0 + 2, 0)
                process(c0 + 1, 1)

            return carry

        lax.fori_loop(0, (nch + 1) // 2, pair, 0)

        if with_max:
            def fin(r, carry):
                for cc in range(8):
                    sacc[r, pl.ds(cc * 16, 16)] = (
                        sacc[r, pl.ds(cc * 16, 16)]
                        * macc[r, pl.ds(cc * 16, 16)])
                return carry

            lax.fori_loop(0, _ROWS, fin, 0)

        pltpu.sync_copy(sacc.at[pl.ds(0, _ROWS)],
                        out_hbm.at[pl.ds(wid * _ROWS, _ROWS)])

    return k


@functools.lru_cache(maxsize=None)
def _make_src_gather():
    @functools.partial(
        pl.kernel,
        mesh=_mesh(),
        compiler_params=_sc_params(),
        out_type=jax.ShapeDtypeStruct((_E, _D), jnp.float32),
        scratch_types=[
            pltpu.VMEM((2, _GCH), jnp.int32),
            pltpu.VMEM((2, _GCH, _D), jnp.float32),
            pltpu.SemaphoreType.DMA,
            pltpu.SemaphoreType.DMA,
        ],
    )
    def k(hnode_hbm, src_hbm, out_hbm, idxb, rows, sem0, sem1):
        base = _wid() * _EPW
        nch = _EPW // _GCH
        sems = (sem0, sem1)

        def issue(c, slot):
            off = base + c * _GCH
            pltpu.sync_copy(src_hbm.at[pl.ds(off, _GCH)], idxb.at[slot])
            pltpu.async_copy(hnode_hbm.at[idxb.at[slot]], rows.at[slot],
                             sems[slot])

        def process(c, slot):
            off = base + c * _GCH
            pltpu.make_async_copy(hnode_hbm.at[idxb.at[slot]],
                                  rows.at[slot], sems[slot]).wait()
            pltpu.sync_copy(rows.at[slot], out_hbm.at[pl.ds(off, _GCH)])

        issue(0, 0)

        def pair(kk, carry):
            c0 = kk * 2

            @pl.when(c0 + 1 < nch)
            def _():
                issue(c0 + 1, 1)
            process(c0, 0)

            @pl.when(c0 + 1 < nch)
            def _():
                @pl.when(c0 + 2 < nch)
                def _():
                    issue(c0 + 2, 0)
                process(c0 + 1, 1)

            return carry

        lax.fori_loop(0, (nch + 1) // 2, pair, 0)

    return k


def _mm_relu(a, w, bm):
    m, kdim = a.shape

    def body(a_ref, w_ref, o_ref):
        o_ref[...] = jax.nn.relu(
            jnp.dot(a_ref[...], w_ref[...],
                    preferred_element_type=jnp.float32))

    return pl.pallas_call(
        body,
        grid=(m // bm,),
        in_specs=[
            pl.BlockSpec((bm, kdim), lambda i: (i, 0)),
            pl.BlockSpec((kdim, _D), lambda i: (0, 0)),
        ],
        out_specs=pl.BlockSpec((bm, _D), lambda i: (i, 0)),
        out_shape=jax.ShapeDtypeStruct((m, _D), jnp.float32),
    )(a, w)


def _node_update(h, msg, w):
    bm = 2048

    def body(h_ref, m_ref, w_ref, o_ref):
        o_ref[...] = jax.nn.relu(
            jnp.dot(h_ref[...] + m_ref[...], w_ref[...],
                    preferred_element_type=jnp.float32))

    return pl.pallas_call(
        body,
        grid=(_NP // bm,),
        in_specs=[
            pl.BlockSpec((bm, _D), lambda i: (i, 0)),
            pl.BlockSpec((bm, _D), lambda i: (i, 0)),
            pl.BlockSpec((_D, _D), lambda i: (0, 0)),
        ],
        out_specs=pl.BlockSpec((bm, _D), lambda i: (i, 0)),
        out_shape=jax.ShapeDtypeStruct((_NP, _D), jnp.float32),
    )(h, msg, w)


def _edge_update(ea, wie, g, e, w):
    bm = 2560

    def body(ea_ref, wie_ref, g_ref, e_ref, w_ref, o_ref):
        e0 = jax.nn.relu(
            jnp.dot(ea_ref[...], wie_ref[...],
                    preferred_element_type=jnp.float32))
        o_ref[...] = jax.nn.relu(
            e0 + jnp.dot(g_ref[...] - e_ref[...], w_ref[...],
                         preferred_element_type=jnp.float32))

    return pl.pallas_call(
        body,
        grid=(_E // bm,),
        in_specs=[
            pl.BlockSpec((bm, _ED), lambda i: (i, 0)),
            pl.BlockSpec((_ED, _D), lambda i: (0, 0)),
            pl.BlockSpec((bm, _D), lambda i: (i, 0)),
            pl.BlockSpec((bm, _D), lambda i: (i, 0)),
            pl.BlockSpec((_D, _D), lambda i: (0, 0)),
        ],
        out_specs=pl.BlockSpec((bm, _D), lambda i: (i, 0)),
        out_shape=jax.ShapeDtypeStruct((_E, _D), jnp.float32),
    )(ea, wie, g, e, w)


def _readout(hn, ag, batch3d, woa, wob, lw, lb):
    bm = 2048
    nblk = _NP // bm

    def body(hn_ref, ag_ref, b_ref, woa_ref, wob_ref, lw_ref, lb_ref,
             out_ref, ssum, scnt):
        i = pl.program_id(0)

        @pl.when(i == 0)
        def _():
            ssum[...] = jnp.zeros_like(ssum)
            scnt[...] = jnp.zeros_like(scnt)

        h = jax.nn.relu(
            jnp.dot(hn_ref[...], woa_ref[...],
                    preferred_element_type=jnp.float32)
            + jnp.dot(ag_ref[...], wob_ref[...],
                      preferred_element_type=jnp.float32))
        bids = b_ref[0, 0, :]
        oh = (bids[:, None]
              == lax.broadcasted_iota(jnp.int32, (1, _G), 1)
              ).astype(jnp.float32)
        dn = (((0,), (0,)), ((), ()))
        ssum[...] += lax.dot_general(oh, h, dn,
                                     preferred_element_type=jnp.float32)
        scnt[...] += lax.dot_general(oh, jnp.ones_like(h), dn,
                                     preferred_element_type=jnp.float32)

        @pl.when(i == nblk - 1)
        def _():
            hm = ssum[...] / jnp.maximum(scnt[...], 1.0)
            out_ref[...] = (
                jnp.dot(hm, lw_ref[...], preferred_element_type=jnp.float32)
                + lb_ref[0, 0])

    return pl.pallas_call(
        body,
        grid=(nblk,),
        in_specs=[
            pl.BlockSpec((bm, _D), lambda i: (i, 0)),
            pl.BlockSpec((bm, _D), lambda i: (i, 0)),
            pl.BlockSpec((1, 1, bm), lambda i: (i, 0, 0)),
            pl.BlockSpec((_D, _D), lambda i: (0, 0)),
            pl.BlockSpec((_D, _D), lambda i: (0, 0)),
            pl.BlockSpec((_D, 1), lambda i: (0, 0)),
            pl.BlockSpec((1, 1), lambda i: (0, 0)),
        ],
        out_specs=pl.BlockSpec((_G, 1), lambda i: (0, 0)),
        out_shape=jax.ShapeDtypeStruct((_G, 1), jnp.float32),
        scratch_shapes=[
            pltpu.VMEM((_G, _D), jnp.float32),
            pltpu.VMEM((_G, _D), jnp.float32),
        ],
    )(hn, ag, batch3d, woa, wob, lw, lb)


def kernel(x, edge_index, edge_attr, batch, W_i_node, W_i_edge, W_comm,
           W_h, W_o, lin_W, lin_b):
    src = edge_index[0].astype(jnp.int32)
    dst = edge_index[1].astype(jnp.int32)
    x_pad = jnp.pad(x, ((0, _NP - _N), (0, 0)))
    batch3d = jnp.pad(batch.astype(jnp.int32), (0, _NP - _N),
                      constant_values=_G).reshape(_NP // 2048, 1, 2048)

    ids, ldst, cnts = _build_lists()(dst)

    h_node = _mm_relu(x_pad, W_i_node, 2048)
    h_edge0 = _mm_relu(edge_attr, W_i_edge, 2560)
    h_edge = h_edge0

    agg_mul = _make_agg(True)
    agg_sum = _make_agg(False)
    src_gather = _make_src_gather()

    for l in range(_L):
        msg = agg_mul(h_edge, ids, ldst, cnts)
        h_node = _node_update(h_node, msg, W_comm[l])
        g = src_gather(h_node, src)
        h_edge = _edge_update(edge_attr, W_i_edge, g, h_edge, W_h[l])

    agg_fin = agg_sum(h_edge, ids, ldst, cnts)
    out = _readout(h_node[:_NP], agg_fin, batch3d,
                   W_o[:_D], W_o[_D:], lin_W, lin_b.reshape(1, 1))
    return out


# speedup vs baseline: 1.1416x; 1.1416x over previous
import functools

import jax
import jax.numpy as jnp
from jax import lax
from jax.experimental import pallas as pl
from jax.experimental.pallas import tpu as pltpu
from jax.experimental.pallas import tpu_sc as plsc

_N = 10000
_E = 320000
_D = 128
_ED = 16
_L = 3
_G = 64

_NW = 32
_NP = 10240
_ROWS = 320
_CAP = 20608
_SCCH = 128
_DCH = 40000
_EPW = _E // _NW
_GCH = 80


def _mesh():
    return plsc.VectorSubcoreMesh(core_axis_name="c", subcore_axis_name="s")


def _sc_params():
    return pltpu.CompilerParams(needs_layout_passes=False)


def _wid():
    return lax.axis_index("s") * 2 + lax.axis_index("c")


@functools.lru_cache(maxsize=None)
def _build_lists():
    @functools.partial(
        pl.kernel,
        mesh=_mesh(),
        compiler_params=_sc_params(),
        out_type=[
            jax.ShapeDtypeStruct((_NW, _CAP), jnp.int32),
            jax.ShapeDtypeStruct((_NW, _CAP), jnp.int32),
            jax.ShapeDtypeStruct((_NW, 16), jnp.int32),
        ],
        scratch_types=[
            pltpu.VMEM((_DCH,), jnp.int32),
            pltpu.VMEM((_CAP,), jnp.int32),
            pltpu.VMEM((_CAP,), jnp.int32),
            pltpu.VMEM((16,), jnp.int32),
        ],
    )
    def k(dst_hbm, ids_hbm, ldst_hbm, cnt_hbm, dchunk, ids_v, ldst_v, cntb):
        wid = _wid()
        lo = wid * _ROWS
        z16 = jnp.zeros((16,), jnp.int32)

        def zb(i, carry):
            ids_v[pl.ds(i * 16, 16)] = z16
            return carry

        lax.fori_loop(0, _CAP // 16, zb, 0)

        def chunk_body(cidx, cnt):
            pltpu.sync_copy(dst_hbm.at[pl.ds(cidx * _DCH, _DCH)], dchunk)

            def ibody(i, cnt):
                v = dchunk[pl.ds(i * 16, 16)]
                eid = (cidx * _DCH + i * 16) + lax.iota(jnp.int32, 16)
                m = (v >= lo) & (v < lo + _ROWS)
                mi = m.astype(jnp.int32)
                pos = cnt + plsc.cumsum(mi) - 1
                plsc.store_scatter(ids_v, [pos], eid, mask=m)
                plsc.store_scatter(ldst_v, [pos], v - lo, mask=m)
                return cnt + jnp.sum(mi)

            return lax.fori_loop(0, _DCH // 16, ibody, cnt)

        cnt = lax.fori_loop(0, _E // _DCH, chunk_body, jnp.int32(0))
        iota16 = lax.iota(jnp.int32, 16)
        for kpad in range(8):
            pos = cnt + kpad * 16 + iota16
            plsc.store_scatter(ids_v, [pos], jnp.zeros((16,), jnp.int32))
            plsc.store_scatter(ldst_v, [pos],
                               jnp.full((16,), _ROWS, jnp.int32))
        cntp = ((cnt + _SCCH - 1) // _SCCH) * _SCCH
        pltpu.sync_copy(ids_v, ids_hbm.at[wid])
        pltpu.sync_copy(ldst_v, ldst_hbm.at[wid])
        cntb[...] = jnp.full((16,), cntp, jnp.int32)
        pltpu.sync_copy(cntb, cnt_hbm.at[wid])

    return k


@functools.lru_cache(maxsize=None)
def _make_agg(with_max):
    @functools.partial(
        pl.kernel,
        mesh=_mesh(),
        compiler_params=_sc_params(),
        out_type=jax.ShapeDtypeStruct((_NP, _D), jnp.float32),
        scratch_types=[
            pltpu.VMEM((_ROWS + 1, _D), jnp.float32),
            pltpu.VMEM((_ROWS + 1, _D), jnp.float32),
            pltpu.VMEM((2, _SCCH), jnp.int32),
            pltpu.VMEM((_SCCH,), jnp.int32),
            pltpu.VMEM((2, _SCCH, _D), jnp.float32),
            pltpu.VMEM((16,), jnp.int32),
            pltpu.SemaphoreType.DMA,
            pltpu.SemaphoreType.DMA,
        ],
    )
    def k(hedge_hbm, ids_hbm, ldst_hbm, cnt_hbm, out_hbm,
          sacc, macc, idsb, ldstb, rows, cntb, sem0, sem1):
        wid = _wid()
        zf = jnp.zeros((16,), jnp.float32)

        def zb(r, carry):
            for cc in range(8):
                sacc[r, pl.ds(cc * 16, 16)] = zf
                if with_max:
                    macc[r, pl.ds(cc * 16, 16)] = zf
            return carry

        lax.fori_loop(0, _ROWS, zb, 0)

        pltpu.sync_copy(cnt_hbm.at[wid], cntb)
        lane = lax.iota(jnp.int32, 16)
        nch = jnp.sum(jnp.where(lane == 0, cntb[...], 0)) // _SCCH
        sems = (sem0, sem1)

        def issue(c, slot):
            pltpu.sync_copy(ids_hbm.at[wid, pl.ds(c * _SCCH, _SCCH)],
                            idsb.at[slot])
            pltpu.async_copy(hedge_hbm.at[idsb.at[slot]], rows.at[slot],
                             sems[slot])

        def process(c, slot):
            base = c * _SCCH
            pltpu.make_async_copy(hedge_hbm.at[idsb.at[slot]],
                                  rows.at[slot], sems[slot]).wait()
            pltpu.sync_copy(ldst_hbm.at[wid, pl.ds(base, _SCCH)], ldstb)

            def grp(g, carry):
                ld16 = ldstb[pl.ds(g * 16, 16)]
                jb = g * 16

                for t in range(16):
                    ld = ld16[t]
                    j = jb + t
                    for r in range(8):
                        a = rows[slot, j, pl.ds(r * 16, 16)]
                        sacc[ld, pl.ds(r * 16, 16)] = (
                            sacc[ld, pl.ds(r * 16, 16)] + a)
                        if with_max:
                            macc[ld, pl.ds(r * 16, 16)] = jnp.maximum(
                                macc[ld, pl.ds(r * 16, 16)], a)

                return carry

            lax.fori_loop(0, _SCCH // 16, grp, 0)

        @pl.when(nch > 0)
        def _():
            issue(0, 0)

        def pair(kk, carry):
            c0 = kk * 2

            @pl.when(c0 + 1 < nch)
            def _():
                issue(c0 + 1, 1)
            process(c0, 0)

            @pl.when(c0 + 1 < nch)
            def _():
                @pl.when(c0 + 2 < nch)
                def _():
                    issue(c0 + 2, 0)
                process(c0 + 1, 1)

            return carry

        lax.fori_loop(0, (nch + 1) // 2, pair, 0)

        if with_max:
            def fin(r, carry):
                for cc in range(8):
                    sacc[r, pl.ds(cc * 16, 16)] = (
                        sacc[r, pl.ds(cc * 16, 16)]
                        * macc[r, pl.ds(cc * 16, 16)])
                return carry

            lax.fori_loop(0, _ROWS, fin, 0)

        pltpu.sync_copy(sacc.at[pl.ds(0, _ROWS)],
                        out_hbm.at[pl.ds(wid * _ROWS, _ROWS)])

    return k


@functools.lru_cache(maxsize=None)
def _make_src_gather():
    @functools.partial(
        pl.kernel,
        mesh=_mesh(),
        compiler_params=_sc_params(),
        out_type=jax.ShapeDtypeStruct((_E, _D), jnp.float32),
        scratch_types=[
            pltpu.VMEM((2, _GCH), jnp.int32),
            pltpu.VMEM((2, _GCH, _D), jnp.float32),
            pltpu.SemaphoreType.DMA,
            pltpu.SemaphoreType.DMA,
        ],
    )
    def k(hnode_hbm, src_hbm, out_hbm, idxb, rows, sem0, sem1):
        base = _wid() * _EPW
        nch = _EPW // _GCH
        sems = (sem0, sem1)

        def issue(c, slot):
            off = base + c * _GCH
            pltpu.sync_copy(src_hbm.at[pl.ds(off, _GCH)], idxb.at[slot])
            pltpu.async_copy(hnode_hbm.at[idxb.at[slot]], rows.at[slot],
                             sems[slot])

        def process(c, slot):
            off = base + c * _GCH
            pltpu.make_async_copy(hnode_hbm.at[idxb.at[slot]],
                                  rows.at[slot], sems[slot]).wait()
            pltpu.sync_copy(rows.at[slot], out_hbm.at[pl.ds(off, _GCH)])

        issue(0, 0)

        def pair(kk, carry):
            c0 = kk * 2

            @pl.when(c0 + 1 < nch)
            def _():
                issue(c0 + 1, 1)
            process(c0, 0)

            @pl.when(c0 + 1 < nch)
            def _():
                @pl.when(c0 + 2 < nch)
                def _():
                    issue(c0 + 2, 0)
                process(c0 + 1, 1)

            return carry

        lax.fori_loop(0, (nch + 1) // 2, pair, 0)

    return k


def _mm_relu(a, w, bm):
    m, kdim = a.shape

    def body(a_ref, w_ref, o_ref):
        o_ref[...] = jax.nn.relu(
            jnp.dot(a_ref[...], w_ref[...],
                    preferred_element_type=jnp.float32))

    return pl.pallas_call(
        body,
        grid=(m // bm,),
        in_specs=[
            pl.BlockSpec((bm, kdim), lambda i: (i, 0)),
            pl.BlockSpec((kdim, _D), lambda i: (0, 0)),
        ],
        out_specs=pl.BlockSpec((bm, _D), lambda i: (i, 0)),
        out_shape=jax.ShapeDtypeStruct((m, _D), jnp.float32),
    )(a, w)


def _node_update(h, msg, w):
    bm = 2048

    def body(h_ref, m_ref, w_ref, o_ref):
        o_ref[...] = jax.nn.relu(
            jnp.dot(h_ref[...] + m_ref[...], w_ref[...],
                    preferred_element_type=jnp.float32))

    return pl.pallas_call(
        body,
        grid=(_NP // bm,),
        in_specs=[
            pl.BlockSpec((bm, _D), lambda i: (i, 0)),
            pl.BlockSpec((bm, _D), lambda i: (i, 0)),
            pl.BlockSpec((_D, _D), lambda i: (0, 0)),
        ],
        out_specs=pl.BlockSpec((bm, _D), lambda i: (i, 0)),
        out_shape=jax.ShapeDtypeStruct((_NP, _D), jnp.float32),
    )(h, msg, w)


def _edge_update(ea, wie, g, e, w):
    bm = 6400

    def body(ea_ref, wie_ref, g_ref, e_ref, w_ref, o_ref):
        e0 = jax.nn.relu(
            jnp.dot(ea_ref[...], wie_ref[...],
                    preferred_element_type=jnp.float32))
        o_ref[...] = jax.nn.relu(
            e0 + jnp.dot(g_ref[...] - e_ref[...], w_ref[...],
                         preferred_element_type=jnp.float32))

    return pl.pallas_call(
        body,
        grid=(_E // bm,),
        in_specs=[
            pl.BlockSpec((bm, _ED), lambda i: (i, 0)),
            pl.BlockSpec((_ED, _D), lambda i: (0, 0)),
            pl.BlockSpec((bm, _D), lambda i: (i, 0)),
            pl.BlockSpec((bm, _D), lambda i: (i, 0)),
            pl.BlockSpec((_D, _D), lambda i: (0, 0)),
        ],
        out_specs=pl.BlockSpec((bm, _D), lambda i: (i, 0)),
        out_shape=jax.ShapeDtypeStruct((_E, _D), jnp.float32),
    )(ea, wie, g, e, w)


def _readout(hn, ag, batch3d, woa, wob, lw, lb):
    bm = 2048
    nblk = _NP // bm

    def body(hn_ref, ag_ref, b_ref, woa_ref, wob_ref, lw_ref, lb_ref,
             out_ref, ssum, scnt):
        i = pl.program_id(0)

        @pl.when(i == 0)
        def _():
            ssum[...] = jnp.zeros_like(ssum)
            scnt[...] = jnp.zeros_like(scnt)

        h = jax.nn.relu(
            jnp.dot(hn_ref[...], woa_ref[...],
                    preferred_element_type=jnp.float32)
            + jnp.dot(ag_ref[...], wob_ref[...],
                      preferred_element_type=jnp.float32))
        bids = b_ref[0, 0, :]
        oh = (bids[:, None]
              == lax.broadcasted_iota(jnp.int32, (1, _G), 1)
              ).astype(jnp.float32)
        dn = (((0,), (0,)), ((), ()))
        ssum[...] += lax.dot_general(oh, h, dn,
                                     preferred_element_type=jnp.float32)
        scnt[...] += lax.dot_general(oh, jnp.ones_like(h), dn,
                                     preferred_element_type=jnp.float32)

        @pl.when(i == nblk - 1)
        def _():
            hm = ssum[...] / jnp.maximum(scnt[...], 1.0)
            out_ref[...] = (
                jnp.dot(hm, lw_ref[...], preferred_element_type=jnp.float32)
                + lb_ref[0, 0])

    return pl.pallas_call(
        body,
        grid=(nblk,),
        in_specs=[
            pl.BlockSpec((bm, _D), lambda i: (i, 0)),
            pl.BlockSpec((bm, _D), lambda i: (i, 0)),
            pl.BlockSpec((1, 1, bm), lambda i: (i, 0, 0)),
            pl.BlockSpec((_D, _D), lambda i: (0, 0)),
            pl.BlockSpec((_D, _D), lambda i: (0, 0)),
            pl.BlockSpec((_D, 1), lambda i: (0, 0)),
            pl.BlockSpec((1, 1), lambda i: (0, 0)),
        ],
        out_specs=pl.BlockSpec((_G, 1), lambda i: (0, 0)),
        out_shape=jax.ShapeDtypeStruct((_G, 1), jnp.float32),
        scratch_shapes=[
            pltpu.VMEM((_G, _D), jnp.float32),
            pltpu.VMEM((_G, _D), jnp.float32),
        ],
    )(hn, ag, batch3d, woa, wob, lw, lb)


def kernel(x, edge_index, edge_attr, batch, W_i_node, W_i_edge, W_comm,
           W_h, W_o, lin_W, lin_b):
    src = edge_index[0].astype(jnp.int32)
    dst = edge_index[1].astype(jnp.int32)
    x_pad = jnp.pad(x, ((0, _NP - _N), (0, 0)))
    batch3d = jnp.pad(batch.astype(jnp.int32), (0, _NP - _N),
                      constant_values=_G).reshape(_NP // 2048, 1, 2048)

    ids, ldst, cnts = _build_lists()(dst)

    h_node = _mm_relu(x_pad, W_i_node, 2048)
    h_edge0 = _mm_relu(edge_attr, W_i_edge, 6400)
    h_edge = h_edge0

    agg_mul = _make_agg(True)
    agg_sum = _make_agg(False)
    src_gather = _make_src_gather()

    for l in range(_L):
        msg = agg_mul(h_edge, ids, ldst, cnts)
        h_node = _node_update(h_node, msg, W_comm[l])
        g = src_gather(h_node, src)
        h_edge = _edge_update(edge_attr, W_i_edge, g, h_edge, W_h[l])

    agg_fin = agg_sum(h_edge, ids, ldst, cnts)
    out = _readout(h_node[:_NP], agg_fin, batch3d,
                   W_o[:_D], W_o[_D:], lin_W, lin_b.reshape(1, 1))
    return out
